# Initial kernel scaffold; baseline (speedup 1.0000x reference)
import jax
import jax.numpy as jnp
from jax.experimental import pallas as pl


def kernel(pos, edge_index, edge_shift, lattice, batch_idx):
    raise NotImplementedError("write your pallas kernel here")


# speedup vs baseline: 19.2846x; 19.2846x over previous
import functools

import jax
import jax.numpy as jnp
from jax import lax
from jax.experimental import pallas as pl
from jax.experimental.pallas import tpu as pltpu
from jax.experimental.pallas import tpu_sc as plsc

N_CORES = 2
N_SUBCORES = 16
N_WORKERS = N_CORES * N_SUBCORES
LANES = 16
CHUNK = 1024
FILL_ROWS = 512
LAT_STRIDE = 16


@functools.lru_cache(maxsize=None)
def _build(n_edges: int, n_nodes: int, n_graphs: int):
    n_chunks = n_edges // CHUNK
    t_max = (n_chunks + N_WORKERS - 1) // N_WORKERS
    n_fill = (n_nodes + FILL_ROWS - 1) // FILL_ROWS
    f_max = (n_fill + N_SUBCORES - 1) // N_SUBCORES
    sp_rows = n_fill * FILL_ROWS
    mesh = plsc.VectorSubcoreMesh(
        core_axis_name="c", subcore_axis_name="s",
        num_cores=N_CORES, num_subcores=N_SUBCORES)

    @functools.partial(
        pl.kernel,
        out_type=jax.ShapeDtypeStruct((n_edges,), jnp.float32),
        mesh=mesh,
        scratch_types=[
            pltpu.VMEM_SHARED((sp_rows, 8), jnp.float32),
            pltpu.VMEM((n_graphs * LAT_STRIDE,), jnp.float32),
            pltpu.VMEM((FILL_ROWS * 8,), jnp.float32),
            pltpu.VMEM((FILL_ROWS, 8), jnp.float32),
            pltpu.VMEM((FILL_ROWS,), jnp.int32),
            pltpu.VMEM((CHUNK,), jnp.int32),
            pltpu.VMEM((CHUNK,), jnp.int32),
            pltpu.VMEM((CHUNK * 3,), jnp.float32),
            pltpu.VMEM((CHUNK, 8), jnp.float32),
            pltpu.VMEM((CHUNK, 8), jnp.float32),
            pltpu.VMEM((CHUNK,), jnp.float32),
            pltpu.SemaphoreType.DMA,
        ],
        compiler_params=pltpu.CompilerParams(
            needs_layout_passes=False, use_tc_tiling_on_sc=False),
    )
    def dist_kernel(ntf, lat, dst, src, sh, out,
                    nt_sp, lat_v, tfill, t2d, fidx, di, si, shv,
                    drow, srow, outv, sem):
        cid = lax.axis_index("c")
        sid = lax.axis_index("s")
        wid = sid * N_CORES + cid

        pltpu.sync_copy(lat, lat_v)

        iota = lax.iota(jnp.int32, LANES)
        lane3 = iota * 3
        iota_d8 = iota >> 3
        iota_m8 = iota & 7
        c0 = jnp.full((LANES,), 0, jnp.int32)
        c1 = jnp.full((LANES,), 1, jnp.int32)
        c2 = jnp.full((LANES,), 2, jnp.int32)
        c3 = jnp.full((LANES,), 3, jnp.int32)

        @pl.loop(0, f_max)
        def _fill(ft):
            fc = sid + ft * N_SUBCORES

            @pl.when(fc < n_fill)
            def _():
                wbase = fc * (FILL_ROWS * 8)
                cp = pltpu.make_async_copy(
                    ntf.at[pl.ds(wbase, FILL_ROWS * 8)], tfill, sem)
                cp.start(); cp.wait()

                @pl.loop(0, FILL_ROWS * 8 // LANES)
                def _re(v):
                    x = tfill[pl.ds(v * LANES, LANES)]
                    rows = iota_d8 + v * (LANES // 8)
                    plsc.store_scatter(t2d, [rows, iota_m8], x)

                @pl.loop(0, FILL_ROWS // LANES)
                def _ix(v):
                    fidx[pl.ds(v * LANES, LANES)] = iota + (fc * FILL_ROWS + v * LANES)

                pltpu.sync_copy(t2d, nt_sp.at[fidx])

        plsc.subcore_barrier()

        @pl.loop(0, t_max)
        def _chunk(t):
            g = wid + t * N_WORKERS

            @pl.when(g < n_chunks)
            def _():
                base = g * CHUNK
                cp_d = pltpu.make_async_copy(dst.at[pl.ds(base, CHUNK)], di, sem)
                cp_s = pltpu.make_async_copy(src.at[pl.ds(base, CHUNK)], si, sem)
                cp_h = pltpu.make_async_copy(
                    sh.at[pl.ds(base * 3, CHUNK * 3)], shv, sem)
                cp_d.start(); cp_s.start(); cp_h.start()
                cp_d.wait(); cp_s.wait(); cp_h.wait()

                gd = pltpu.make_async_copy(nt_sp.at[di], drow, sem)
                gs = pltpu.make_async_copy(nt_sp.at[si], srow, sem)
                gd.start(); gs.start()
                gd.wait(); gs.wait()

                @pl.loop(0, CHUNK // LANES, unroll=4)
                def _vreg(v):
                    rows = iota + v * LANES
                    sx = plsc.load_gather(srow, [rows, c0])
                    sy = plsc.load_gather(srow, [rows, c1])
                    sz = plsc.load_gather(srow, [rows, c2])
                    bf = plsc.load_gather(srow, [rows, c3])
                    dx = plsc.load_gather(drow, [rows, c0])
                    dy = plsc.load_gather(drow, [rows, c1])
                    dz = plsc.load_gather(drow, [rows, c2])
                    shb = lane3 + v * (LANES * 3)
                    s0 = plsc.load_gather(shv, [shb])
                    s1 = plsc.load_gather(shv, [shb + 1])
                    s2 = plsc.load_gather(shv, [shb + 2])
                    bb = bf.astype(jnp.int32) << 4
                    l00 = plsc.load_gather(lat_v, [bb + 0])
                    l01 = plsc.load_gather(lat_v, [bb + 1])
                    l02 = plsc.load_gather(lat_v, [bb + 2])
                    l10 = plsc.load_gather(lat_v, [bb + 3])
                    l11 = plsc.load_gather(lat_v, [bb + 4])
                    l12 = plsc.load_gather(lat_v, [bb + 5])
                    l20 = plsc.load_gather(lat_v, [bb + 6])
                    l21 = plsc.load_gather(lat_v, [bb + 7])
                    l22 = plsc.load_gather(lat_v, [bb + 8])
                    vx = dx - sx + (s0 * l00 + s1 * l10 + s2 * l20)
                    vy = dy - sy + (s0 * l01 + s1 * l11 + s2 * l21)
                    vz = dz - sz + (s0 * l02 + s1 * l12 + s2 * l22)
                    ssq = vx * vx + vy * vy + vz * vz
                    yi = jnp.int32(0x5F3759DF) - (plsc.bitcast(ssq, jnp.int32) >> 1)
                    y = plsc.bitcast(yi, jnp.float32)
                    y = y * (1.5 - 0.5 * ssq * y * y)
                    y = y * (1.5 - 0.5 * ssq * y * y)
                    outv[pl.ds(v * LANES, LANES)] = ssq * y

                pltpu.sync_copy(outv, out.at[pl.ds(base, CHUNK)])

    return dist_kernel


def kernel(pos, edge_index, edge_shift, lattice, batch_idx):
    n_edges = edge_index.shape[1]
    n_nodes = pos.shape[0]
    n_graphs = lattice.shape[0]
    n_fill = (n_nodes + FILL_ROWS - 1) // FILL_ROWS
    pad = n_fill * FILL_ROWS - n_nodes
    nt = jnp.concatenate(
        [pos, batch_idx.astype(jnp.float32)[:, None],
         jnp.zeros((n_nodes, 4), jnp.float32)], axis=1)
    ntf = jnp.pad(nt.reshape(-1), (0, pad * 8))
    lat_flat = jnp.pad(
        lattice.reshape(n_graphs, 9), ((0, 0), (0, LAT_STRIDE - 9))).reshape(-1)
    return _build(n_edges, n_nodes, n_graphs)(
        ntf, lat_flat, edge_index[0], edge_index[1], edge_shift.reshape(-1))


# speedup vs baseline: 19.9199x; 1.0329x over previous
import functools

import jax
import jax.numpy as jnp
from jax import lax
from jax.experimental import pallas as pl
from jax.experimental.pallas import tpu as pltpu
from jax.experimental.pallas import tpu_sc as plsc

N_CORES = 2
N_SUBCORES = 16
N_WORKERS = N_CORES * N_SUBCORES
LANES = 16
CHUNK = 1024
FILL_ROWS = 512
LAT_STRIDE = 16


@functools.lru_cache(maxsize=None)
def _build(n_edges: int, n_nodes: int, n_graphs: int):
    n_chunks = n_edges // CHUNK
    t_max = (n_chunks + N_WORKERS - 1) // N_WORKERS
    n_fill = (n_nodes + FILL_ROWS - 1) // FILL_ROWS
    f_max = (n_fill + N_SUBCORES - 1) // N_SUBCORES
    sp_rows = n_fill * FILL_ROWS
    mesh = plsc.VectorSubcoreMesh(
        core_axis_name="c", subcore_axis_name="s",
        num_cores=N_CORES, num_subcores=N_SUBCORES)

    @functools.partial(
        pl.kernel,
        out_type=jax.ShapeDtypeStruct((n_edges,), jnp.float32),
        mesh=mesh,
        scratch_types=[
            pltpu.VMEM_SHARED((sp_rows, 8), jnp.float32),
            pltpu.VMEM((n_graphs * LAT_STRIDE,), jnp.float32),
            pltpu.VMEM((FILL_ROWS * 8,), jnp.float32),
            pltpu.VMEM((FILL_ROWS, 8), jnp.float32),
            pltpu.VMEM((FILL_ROWS,), jnp.int32),
            pltpu.VMEM((CHUNK,), jnp.int32),
            pltpu.VMEM((CHUNK,), jnp.int32),
            pltpu.VMEM((CHUNK,), jnp.int32),
            pltpu.VMEM((CHUNK,), jnp.int32),
            pltpu.VMEM((CHUNK * 3,), jnp.float32),
            pltpu.VMEM((CHUNK * 3,), jnp.float32),
            pltpu.VMEM((CHUNK, 8), jnp.float32),
            pltpu.VMEM((CHUNK, 8), jnp.float32),
            pltpu.VMEM((CHUNK, 8), jnp.float32),
            pltpu.VMEM((CHUNK, 8), jnp.float32),
            pltpu.VMEM((CHUNK,), jnp.float32),
            pltpu.VMEM((CHUNK,), jnp.float32),
            pltpu.SemaphoreType.DMA,
            pltpu.SemaphoreType.DMA,
            pltpu.SemaphoreType.DMA,
            pltpu.SemaphoreType.DMA,
            pltpu.SemaphoreType.DMA,
            pltpu.SemaphoreType.DMA,
            pltpu.SemaphoreType.DMA,
            pltpu.SemaphoreType.DMA,
            pltpu.SemaphoreType.DMA,
        ],
        compiler_params=pltpu.CompilerParams(
            needs_layout_passes=False, use_tc_tiling_on_sc=False),
    )
    def dist_kernel(ntf, lat, dst, src, sh, out,
                    nt_sp, lat_v, tfill, t2d, fidx,
                    di0, di1, si0, si1, shv0, shv1,
                    drow0, drow1, srow0, srow1, outv0, outv1,
                    sem, isem0, isem1, hsem0, hsem1,
                    gsem0, gsem1, osem0, osem1):
        di = [di0, di1]; si = [si0, si1]; shv = [shv0, shv1]
        drow = [drow0, drow1]; srow = [srow0, srow1]; outv = [outv0, outv1]
        isem = [isem0, isem1]; hsem = [hsem0, hsem1]
        gsem = [gsem0, gsem1]; osem = [osem0, osem1]
        cid = lax.axis_index("c")
        sid = lax.axis_index("s")
        wid = sid * N_CORES + cid

        pltpu.sync_copy(lat, lat_v)

        iota = lax.iota(jnp.int32, LANES)
        lane3 = iota * 3
        iota_d8 = iota >> 3
        iota_m8 = iota & 7
        c0 = jnp.full((LANES,), 0, jnp.int32)
        c1 = jnp.full((LANES,), 1, jnp.int32)
        c2 = jnp.full((LANES,), 2, jnp.int32)
        c3 = jnp.full((LANES,), 3, jnp.int32)

        @pl.loop(0, f_max)
        def _fill(ft):
            fc = sid + ft * N_SUBCORES

            @pl.when(fc < n_fill)
            def _():
                wbase = fc * (FILL_ROWS * 8)
                cp = pltpu.make_async_copy(
                    ntf.at[pl.ds(wbase, FILL_ROWS * 8)], tfill, sem)
                cp.start(); cp.wait()

                @pl.loop(0, FILL_ROWS * 8 // LANES)
                def _re(v):
                    x = tfill[pl.ds(v * LANES, LANES)]
                    rows = iota_d8 + v * (LANES // 8)
                    plsc.store_scatter(t2d, [rows, iota_m8], x)

                @pl.loop(0, FILL_ROWS // LANES)
                def _ix(v):
                    fidx[pl.ds(v * LANES, LANES)] = iota + (fc * FILL_ROWS + v * LANES)

                pltpu.sync_copy(t2d, nt_sp.at[fidx])

        plsc.subcore_barrier()

        def cbase(t):
            return (wid + t * N_WORKERS) * CHUNK

        def d_idx(t, p):
            b = cbase(t)
            return (pltpu.make_async_copy(dst.at[pl.ds(b, CHUNK)], di[p], isem[p]),
                    pltpu.make_async_copy(src.at[pl.ds(b, CHUNK)], si[p], isem[p]))

        def d_sh(t, p):
            b = cbase(t)
            return pltpu.make_async_copy(
                sh.at[pl.ds(b * 3, CHUNK * 3)], shv[p], hsem[p])

        def d_gath(p):
            return (pltpu.make_async_copy(nt_sp.at[di[p]], drow[p], gsem[p]),
                    pltpu.make_async_copy(nt_sp.at[si[p]], srow[p], gsem[p]))

        def d_out(t, p):
            return pltpu.make_async_copy(
                outv[p], out.at[pl.ds(cbase(t), CHUNK)], osem[p])

        def compute(p):
            dr, sr, hv, ov = drow[p], srow[p], shv[p], outv[p]

            @pl.loop(0, CHUNK // LANES, unroll=4)
            def _vreg(v):
                rows = iota + v * LANES
                sx = plsc.load_gather(sr, [rows, c0])
                sy = plsc.load_gather(sr, [rows, c1])
                sz = plsc.load_gather(sr, [rows, c2])
                bf = plsc.load_gather(sr, [rows, c3])
                dx = plsc.load_gather(dr, [rows, c0])
                dy = plsc.load_gather(dr, [rows, c1])
                dz = plsc.load_gather(dr, [rows, c2])
                shb = lane3 + v * (LANES * 3)
                s0 = plsc.load_gather(hv, [shb])
                s1 = plsc.load_gather(hv, [shb + 1])
                s2 = plsc.load_gather(hv, [shb + 2])
                bb = bf.astype(jnp.int32) << 4
                l00 = plsc.load_gather(lat_v, [bb + 0])
                l01 = plsc.load_gather(lat_v, [bb + 1])
                l02 = plsc.load_gather(lat_v, [bb + 2])
                l10 = plsc.load_gather(lat_v, [bb + 3])
                l11 = plsc.load_gather(lat_v, [bb + 4])
                l12 = plsc.load_gather(lat_v, [bb + 5])
                l20 = plsc.load_gather(lat_v, [bb + 6])
                l21 = plsc.load_gather(lat_v, [bb + 7])
                l22 = plsc.load_gather(lat_v, [bb + 8])
                vx = dx - sx + (s0 * l00 + s1 * l10 + s2 * l20)
                vy = dy - sy + (s0 * l01 + s1 * l11 + s2 * l21)
                vz = dz - sz + (s0 * l02 + s1 * l12 + s2 * l22)
                ssq = vx * vx + vy * vy + vz * vz
                yi = jnp.int32(0x5F3759DF) - (plsc.bitcast(ssq, jnp.int32) >> 1)
                y = plsc.bitcast(yi, jnp.float32)
                y = y * (1.5 - 0.5 * ssq * y * y)
                y = y * (1.5 - 0.5 * ssq * y * y)
                ov[pl.ds(v * LANES, LANES)] = ssq * y

        n_lim = n_chunks

        a0, b0 = d_idx(0, 0)
        a0.start(); b0.start()
        d_sh(0, 0).start()
        a0.wait(); b0.wait()
        g0a, g0b = d_gath(0)
        g0a.start(); g0b.start()

        @pl.when(wid + N_WORKERS < n_lim)
        def _p1():
            a1, b1 = d_idx(1, 1)
            a1.start(); b1.start()
            d_sh(1, 1).start()

        def body(t, p):
            q = 1 - p
            g = wid + t * N_WORKERS

            @pl.when(g < n_lim)
            def _():
                ga, gb = d_gath(p)
                ga.wait(); gb.wait()

                @pl.when(g + N_WORKERS < n_lim)
                def _fg():
                    ia, ib = d_idx(t + 1, q)
                    ia.wait(); ib.wait()
                    na, nb = d_gath(q)
                    na.start(); nb.start()

                @pl.when(g + 2 * N_WORKERS < n_lim)
                def _fi():
                    ia, ib = d_idx(t + 2, p)
                    ia.start(); ib.start()

                d_sh(t, p).wait()

                @pl.when(t >= 2)
                def _wo():
                    d_out(t - 2, p).wait()

                compute(p)
                d_out(t, p).start()

                @pl.when(g + 2 * N_WORKERS < n_lim)
                def _fh():
                    d_sh(t + 2, p).start()

        @pl.loop(0, t_max // 2)
        def _pair(u):
            t0 = u * 2
            body(t0, 0)
            body(t0 + 1, 1)

        if t_max % 2:
            body(t_max - 1, (t_max - 1) % 2)

        for k in (t_max - 3, t_max - 2, t_max - 1):
            if k >= 0:
                gk = wid + k * N_WORKERS

                @pl.when((gk < n_lim) & (gk + 2 * N_WORKERS >= n_lim))
                def _d(k=k):
                    d_out(k, k % 2).wait()

    return dist_kernel


def kernel(pos, edge_index, edge_shift, lattice, batch_idx):
    n_edges = edge_index.shape[1]
    n_nodes = pos.shape[0]
    n_graphs = lattice.shape[0]
    n_fill = (n_nodes + FILL_ROWS - 1) // FILL_ROWS
    pad = n_fill * FILL_ROWS - n_nodes
    nt = jnp.concatenate(
        [pos, batch_idx.astype(jnp.float32)[:, None],
         jnp.zeros((n_nodes, 4), jnp.float32)], axis=1)
    ntf = jnp.pad(nt.reshape(-1), (0, pad * 8))
    lat_flat = jnp.pad(
        lattice.reshape(n_graphs, 9), ((0, 0), (0, LAT_STRIDE - 9))).reshape(-1)
    return _build(n_edges, n_nodes, n_graphs)(
        ntf, lat_flat, edge_index[0], edge_index[1], edge_shift.reshape(-1))
